# baseline (speedup 1.0000x reference)
import functools

import jax
import jax.numpy as jnp
from jax import lax
from jax.experimental import pallas as pl
from jax.experimental.pallas import tpu as pltpu
from jax.experimental.pallas import tpu_sc as plsc

_NC = 2
_NS = 16
_NW = _NC * _NS
_CH = 128
_K = 16


def _mm_body(x_ref, w_ref, o_ref):
    o_ref[...] = jnp.dot(x_ref[...], w_ref[...], preferred_element_type=jnp.float32)


def _matmul(x, w, block_rows=1000):
    n, d = x.shape
    dout = w.shape[1]
    grid = (n // block_rows,)
    return pl.pallas_call(
        _mm_body,
        grid=grid,
        in_specs=[
            pl.BlockSpec((block_rows, d), lambda i: (i, 0)),
            pl.BlockSpec((d, dout), lambda i: (0, 0)),
        ],
        out_specs=pl.BlockSpec((block_rows, dout), lambda i: (i, 0)),
        out_shape=jax.ShapeDtypeStruct((n, dout), jnp.float32),
    )(x, w)


def _norm_mm_body(p_ref, deg_ref, b_ref, w_ref, o_ref):
    agg = p_ref[0] + p_ref[1]
    deg = jnp.sum(deg_ref[...], axis=0)[:, None]
    inv = 1.0 / jnp.maximum(deg, 1.0)
    h = jnp.maximum(agg * inv + b_ref[...], 0.0)
    o_ref[...] = jnp.dot(h, w_ref[...], preferred_element_type=jnp.float32)


def _norm_matmul(p, deg_all, b, w, block_rows=1024):
    npad, d = p.shape[1], p.shape[2]
    dout = w.shape[1]
    grid = (npad // block_rows,)
    return pl.pallas_call(
        _norm_mm_body,
        grid=grid,
        in_specs=[
            pl.BlockSpec((2, block_rows, d), lambda i: (0, i, 0)),
            pl.BlockSpec((_NW, block_rows), lambda i: (0, i)),
            pl.BlockSpec((1, d), lambda i: (0, 0)),
            pl.BlockSpec((d, dout), lambda i: (0, 0)),
        ],
        out_specs=pl.BlockSpec((block_rows, dout), lambda i: (i, 0)),
        out_shape=jax.ShapeDtypeStruct((npad, dout), jnp.float32),
    )(p, deg_all, b, w)


def _make_heads_body(n):
    def _heads_body(q_ref, deg_ref, b1_ref, wpg_ref, wpd_ref, wv_ref,
                    bpg_ref, bpd_ref, bv_ref, pi_ref, v_ref):
        agg = q_ref[0, pl.ds(0, n), :] + q_ref[1, pl.ds(0, n), :]
        deg = jnp.sum(deg_ref[:, pl.ds(0, n)], axis=0)[:, None]
        inv = 1.0 / jnp.maximum(deg, 1.0)
        h2 = jnp.maximum(agg * inv + b1_ref[...], 0.0)
        mn = jnp.mean(h2, axis=0, keepdims=True)
        pi_ref[pl.ds(0, n), :] = (
            jnp.dot(h2, wpg_ref[...], preferred_element_type=jnp.float32) + bpg_ref[...]
        )
        pi_ref[pl.ds(n, 1), :] = (
            jnp.dot(mn, wpd_ref[...], preferred_element_type=jnp.float32) + bpd_ref[...]
        )
        v_ref[...] = jnp.dot(mn, wv_ref[...], preferred_element_type=jnp.float32) + bv_ref[...]
    return _heads_body


def _heads(n, q, deg_all, b1, wpg, wpd, wv, bpg, bpd, bv):
    return pl.pallas_call(
        _make_heads_body(n),
        out_shape=(
            jax.ShapeDtypeStruct((n + 1, 1), jnp.float32),
            jax.ShapeDtypeStruct((1, 1), jnp.float32),
        ),
    )(q, deg_all, b1, wpg, wpd, wv, bpg, bpd, bv)


@functools.lru_cache(maxsize=None)
def _make_deg(npad, nchunks):
    nstage = nchunks // _K
    assert nchunks % _K == 0

    mesh = plsc.VectorSubcoreMesh(core_axis_name="c", subcore_axis_name="s")

    def body(dst_hbm, deg_hbm, dst_v, deg_l):
        cid = lax.axis_index("c")
        sid = lax.axis_index("s")
        wid = sid * _NC + cid

        zeros16 = jnp.zeros((16,), jnp.float32)
        ones16 = jnp.ones((16,), jnp.float32)

        def z_body(i, _):
            deg_l[pl.ds(i * 16, 16)] = zeros16
            return 0
        lax.fori_loop(0, npad // 16, z_body, 0)

        def s_body(s, _):
            pltpu.sync_copy(dst_hbm.at[wid, pl.ds(s * _K, _K)], dst_v)

            def c_body(j, _):
                for l in range(_CH // 16):
                    idxv = dst_v[j, pl.ds(l * 16, 16)]
                    plsc.addupdate_scatter(deg_l, [idxv], ones16)
                return 0
            lax.fori_loop(0, _K, c_body, 0)
            return 0
        lax.fori_loop(0, nstage, s_body, 0)

        pltpu.sync_copy(deg_l, deg_hbm.at[wid])

    return pl.kernel(
        body,
        out_type=jax.ShapeDtypeStruct((_NW, npad), jnp.float32),
        mesh=mesh,
        compiler_params=pltpu.CompilerParams(
            use_tc_tiling_on_sc=False, needs_layout_passes=False),
        scratch_types=[
            pltpu.VMEM((_K, _CH), jnp.int32),
            pltpu.VMEM((npad,), jnp.float32),
        ],
    )


@functools.lru_cache(maxsize=None)
def _make_mp(npad, nchunks, d):
    rows_per_tile = npad // _NS
    n_wb = rows_per_tile // _CH
    assert rows_per_tile % _CH == 0 and nchunks % (2 * _K) == 0
    npair = nchunks // 2

    mesh = plsc.VectorSubcoreMesh(core_axis_name="c", subcore_axis_name="s")

    def body(hw_hbm, src_hbm, dst_hbm, part_hbm,
             src_v, dst_v, rows_v, agg_sh, g0, g1, s0, s1):
        cid = lax.axis_index("c")
        sid = lax.axis_index("s")
        wid = sid * _NC + cid

        zeros16 = jnp.zeros((16,), jnp.float32)

        def z_body(i, _):
            def z_inner(k, _):
                rows_v[0, i, pl.ds(k * 16, 16)] = zeros16
                return 0
            lax.fori_loop(0, d // 16, z_inner, 0)
            return 0
        lax.fori_loop(0, _CH, z_body, 0)
        for k in range(n_wb):
            r0 = sid * rows_per_tile + k * _CH
            pltpu.sync_copy(rows_v.at[0], agg_sh.at[pl.ds(r0, _CH)])
        plsc.subcore_barrier()

        def wait_gather(buf, sem):
            pltpu.make_async_copy(hw_hbm.at[pl.ds(0, _CH)], rows_v.at[buf], sem).wait()

        def wait_scatter(buf, sem):
            pltpu.make_async_copy(rows_v.at[buf], agg_sh.at[pl.ds(0, _CH)], sem).wait()

        def pair(h, _):
            j0 = 2 * h
            st = j0 // _K
            slot = lax.rem(st, 2)
            k0 = lax.rem(j0, _K)
            jm1 = jnp.maximum(j0 - 1, 0)
            slot_m1 = lax.rem(jm1 // _K, 2)
            k_m1 = lax.rem(jm1, _K)

            @pl.when(lax.rem(j0, _K) == 0)
            def _load():
                pltpu.sync_copy(src_hbm.at[wid, pl.ds(st * _K, _K)], src_v.at[slot])
                pltpu.sync_copy(dst_hbm.at[wid, pl.ds(st * _K, _K)], dst_v.at[slot])

            @pl.when(h > 0)
            def _w0():
                wait_scatter(0, s0)
            pltpu.async_copy(hw_hbm.at[src_v.at[slot, k0]], rows_v.at[0], g0)

            @pl.when(h > 0)
            def _s1():
                wait_gather(1, g1)
                pltpu.async_copy(
                    rows_v.at[1], agg_sh.at[dst_v.at[slot_m1, k_m1]], s1, add=True)
                wait_scatter(1, s1)

            pltpu.async_copy(hw_hbm.at[src_v.at[slot, k0 + 1]], rows_v.at[1], g1)
            wait_gather(0, g0)
            pltpu.async_copy(
                rows_v.at[0], agg_sh.at[dst_v.at[slot, k0]], s0, add=True)
            return 0
        lax.fori_loop(0, npair, pair, 0)

        slot_l = ((nchunks - 1) // _K) % 2
        k_l = (nchunks - 1) % _K
        wait_gather(1, g1)
        pltpu.async_copy(rows_v.at[1], agg_sh.at[dst_v.at[slot_l, k_l]], s1, add=True)
        wait_scatter(0, s0)
        wait_scatter(1, s1)
        plsc.subcore_barrier()

        for k in range(n_wb):
            b = k % 2
            r0 = sid * rows_per_tile + k * _CH
            if k >= 2:
                pltpu.make_async_copy(
                    rows_v.at[b], part_hbm.at[cid, pl.ds(0, _CH)], (g0, g1)[b]).wait()
            pltpu.sync_copy(agg_sh.at[pl.ds(r0, _CH)], rows_v.at[b])
            pltpu.async_copy(rows_v.at[b], part_hbm.at[cid, pl.ds(r0, _CH)], (g0, g1)[b])
        for k in range(max(n_wb - 2, 0), n_wb):
            b = k % 2
            pltpu.make_async_copy(
                rows_v.at[b], part_hbm.at[cid, pl.ds(0, _CH)], (g0, g1)[b]).wait()

    return pl.kernel(
        body,
        out_type=jax.ShapeDtypeStruct((_NC, npad, d), jnp.float32),
        mesh=mesh,
        compiler_params=pltpu.CompilerParams(use_tc_tiling_on_sc=False),
        scratch_types=[
            pltpu.VMEM((2, _K, _CH), jnp.int32),
            pltpu.VMEM((2, _K, _CH), jnp.int32),
            pltpu.VMEM((2, _CH, d), jnp.float32),
            pltpu.VMEM_SHARED((npad, d), jnp.float32),
            pltpu.SemaphoreType.DMA,
            pltpu.SemaphoreType.DMA,
            pltpu.SemaphoreType.DMA,
            pltpu.SemaphoreType.DMA,
        ],
    )


def kernel(x, edge_index, W0, b0, W1, b1, Wpg, bpg, Wpd, bpd, Wv, bv):
    n, d = x.shape
    e = edge_index.shape[1]

    nchunks = -(-(-(-e // (_NW * _CH))) // (2 * _K)) * (2 * _K)
    epw = nchunks * _CH
    e_pad = _NW * epw
    npad = -(-(n + 1) // (_NS * _CH)) * (_NS * _CH)

    pad = e_pad - e
    src = jnp.concatenate([edge_index[0], jnp.zeros((pad,), jnp.int32)])
    dst = jnp.concatenate([edge_index[1], jnp.full((pad,), n, jnp.int32)])
    src = src.reshape(_NW, nchunks, _CH)
    dst = dst.reshape(_NW, nchunks, _CH)

    deg_all = _make_deg(npad, nchunks)(dst)
    mp = _make_mp(npad, nchunks, d)

    xw0 = jnp.pad(_matmul(x, W0), ((0, npad - n), (0, 0)))
    p1 = mp(xw0, src, dst)
    h1w1 = _norm_matmul(p1, deg_all, b0.reshape(1, d), W1)
    p2 = mp(h1w1, src, dst)
    pi, v = _heads(
        n, p2, deg_all, b1.reshape(1, d),
        Wpg, Wpd, Wv,
        bpg.reshape(1, 1), bpd.reshape(1, 1), bv.reshape(1, 1),
    )
    return (pi, v)


# speedup vs baseline: 1.0010x; 1.0010x over previous
import functools

import jax
import jax.numpy as jnp
from jax import lax
from jax.experimental import pallas as pl
from jax.experimental.pallas import tpu as pltpu
from jax.experimental.pallas import tpu_sc as plsc

_NC = 2
_NS = 16
_NW = _NC * _NS
_CH = 128
_K = 16


def _gcn_body(p_ref, deg_ref, b_ref, w_ref, o_ref):
    agg = p_ref[0] + p_ref[1]
    deg = jnp.sum(deg_ref[...], axis=0)[:, None]
    agg = agg / jnp.maximum(deg, 1.0)
    o_ref[...] = jnp.maximum(
        jnp.dot(agg, w_ref[...], preferred_element_type=jnp.float32) + b_ref[...],
        0.0,
    )


def _gcn_dense(p, deg_all, w, b, block_rows=1024):
    npad, d = p.shape[1], p.shape[2]
    dout = w.shape[1]
    grid = (npad // block_rows,)
    return pl.pallas_call(
        _gcn_body,
        grid=grid,
        in_specs=[
            pl.BlockSpec((2, block_rows, d), lambda i: (0, i, 0)),
            pl.BlockSpec((_NW, block_rows), lambda i: (0, i)),
            pl.BlockSpec((1, dout), lambda i: (0, 0)),
            pl.BlockSpec((d, dout), lambda i: (0, 0)),
        ],
        out_specs=pl.BlockSpec((block_rows, dout), lambda i: (i, 0)),
        out_shape=jax.ShapeDtypeStruct((npad, dout), jnp.float32),
    )(p, deg_all, b, w)


def _make_heads_body(n):
    def _heads_body(q_ref, deg_ref, b1_ref, w1_ref, wpg_ref, wpd_ref, wv_ref,
                    bpg_ref, bpd_ref, bv_ref, pi_ref, v_ref):
        agg = q_ref[0, pl.ds(0, n), :] + q_ref[1, pl.ds(0, n), :]
        deg = jnp.sum(deg_ref[:, pl.ds(0, n)], axis=0)[:, None]
        agg = agg / jnp.maximum(deg, 1.0)
        h2 = jnp.maximum(
            jnp.dot(agg, w1_ref[...], preferred_element_type=jnp.float32) + b1_ref[...],
            0.0,
        )
        mn = jnp.mean(h2, axis=0, keepdims=True)
        pi_ref[pl.ds(0, n), :] = (
            jnp.dot(h2, wpg_ref[...], preferred_element_type=jnp.float32) + bpg_ref[...]
        )
        pi_ref[pl.ds(n, 1), :] = (
            jnp.dot(mn, wpd_ref[...], preferred_element_type=jnp.float32) + bpd_ref[...]
        )
        v_ref[...] = jnp.dot(mn, wv_ref[...], preferred_element_type=jnp.float32) + bv_ref[...]
    return _heads_body


def _heads(n, q, deg_all, b1, w1, wpg, wpd, wv, bpg, bpd, bv):
    return pl.pallas_call(
        _make_heads_body(n),
        out_shape=(
            jax.ShapeDtypeStruct((n + 1, 1), jnp.float32),
            jax.ShapeDtypeStruct((1, 1), jnp.float32),
        ),
    )(q, deg_all, b1, w1, wpg, wpd, wv, bpg, bpd, bv)


@functools.lru_cache(maxsize=None)
def _make_deg(npad, nchunks):
    nstage = nchunks // _K
    assert nchunks % _K == 0

    mesh = plsc.VectorSubcoreMesh(core_axis_name="c", subcore_axis_name="s")

    def body(dst_hbm, deg_hbm, dst_v, deg_l):
        cid = lax.axis_index("c")
        sid = lax.axis_index("s")
        wid = sid * _NC + cid

        zeros16 = jnp.zeros((16,), jnp.float32)
        ones16 = jnp.ones((16,), jnp.float32)

        def z_body(i, _):
            deg_l[pl.ds(i * 16, 16)] = zeros16
            return 0
        lax.fori_loop(0, npad // 16, z_body, 0)

        def s_body(s, _):
            pltpu.sync_copy(dst_hbm.at[wid, pl.ds(s * _K, _K)], dst_v)

            def c_body(j, _):
                for l in range(_CH // 16):
                    idxv = dst_v[j, pl.ds(l * 16, 16)]
                    plsc.addupdate_scatter(deg_l, [idxv], ones16)
                return 0
            lax.fori_loop(0, _K, c_body, 0)
            return 0
        lax.fori_loop(0, nstage, s_body, 0)

        pltpu.sync_copy(deg_l, deg_hbm.at[wid])

    return pl.kernel(
        body,
        out_type=jax.ShapeDtypeStruct((_NW, npad), jnp.float32),
        mesh=mesh,
        compiler_params=pltpu.CompilerParams(
            use_tc_tiling_on_sc=False, needs_layout_passes=False),
        scratch_types=[
            pltpu.VMEM((_K, _CH), jnp.int32),
            pltpu.VMEM((npad,), jnp.float32),
        ],
    )


@functools.lru_cache(maxsize=None)
def _make_mp(npad, nchunks, d):
    rows_per_tile = npad // _NS
    n_wb = rows_per_tile // _CH
    assert rows_per_tile % _CH == 0 and nchunks % (2 * _K) == 0
    npair = nchunks // 2

    mesh = plsc.VectorSubcoreMesh(core_axis_name="c", subcore_axis_name="s")

    def body(hw_hbm, src_hbm, dst_hbm, part_hbm,
             src_v, dst_v, rows_v, agg_sh, g0, g1, s0, s1):
        cid = lax.axis_index("c")
        sid = lax.axis_index("s")
        wid = sid * _NC + cid

        zeros16 = jnp.zeros((16,), jnp.float32)

        def z_body(i, _):
            def z_inner(k, _):
                rows_v[0, i, pl.ds(k * 16, 16)] = zeros16
                return 0
            lax.fori_loop(0, d // 16, z_inner, 0)
            return 0
        lax.fori_loop(0, _CH, z_body, 0)
        for k in range(n_wb):
            r0 = sid * rows_per_tile + k * _CH
            pltpu.sync_copy(rows_v.at[0], agg_sh.at[pl.ds(r0, _CH)])
        plsc.subcore_barrier()

        nstage = nchunks // _K

        def s_body(s, _):
            pltpu.sync_copy(src_hbm.at[wid, pl.ds(s * _K, _K)], src_v.at[0])
            pltpu.sync_copy(dst_hbm.at[wid, pl.ds(s * _K, _K)], dst_v.at[0])

            def e_body(j, _):
                pltpu.async_copy(hw_hbm.at[src_v.at[0, j]], rows_v.at[0], g0).wait()
                pltpu.sync_copy(rows_v.at[0], agg_sh.at[dst_v.at[0, j]], add=True)
                return 0
            lax.fori_loop(0, _K, e_body, 0)
            return 0
        lax.fori_loop(0, nstage, s_body, 0)
        plsc.subcore_barrier()

        for k in range(n_wb):
            r0 = sid * rows_per_tile + k * _CH
            pltpu.sync_copy(agg_sh.at[pl.ds(r0, _CH)], rows_v.at[0])
            pltpu.sync_copy(rows_v.at[0], part_hbm.at[cid, pl.ds(r0, _CH)])

    return pl.kernel(
        body,
        out_type=jax.ShapeDtypeStruct((_NC, npad, d), jnp.float32),
        mesh=mesh,
        compiler_params=pltpu.CompilerParams(use_tc_tiling_on_sc=False),
        scratch_types=[
            pltpu.VMEM((2, _K, _CH), jnp.int32),
            pltpu.VMEM((2, _K, _CH), jnp.int32),
            pltpu.VMEM((2, _CH, d), jnp.float32),
            pltpu.VMEM_SHARED((npad, d), jnp.float32),
            pltpu.SemaphoreType.DMA,
            pltpu.SemaphoreType.DMA,
            pltpu.SemaphoreType.DMA,
            pltpu.SemaphoreType.DMA,
        ],
    )


def kernel(x, edge_index, W0, b0, W1, b1, Wpg, bpg, Wpd, bpd, Wv, bv):
    n, d = x.shape
    e = edge_index.shape[1]

    nchunks = -(-(-(-e // (_NW * _CH))) // (2 * _K)) * (2 * _K)
    epw = nchunks * _CH
    e_pad = _NW * epw
    npad = -(-(n + 1) // (_NS * _CH)) * (_NS * _CH)

    pad = e_pad - e
    src = jnp.concatenate([edge_index[0], jnp.zeros((pad,), jnp.int32)])
    dst = jnp.concatenate([edge_index[1], jnp.full((pad,), n, jnp.int32)])
    src = src.reshape(_NW, nchunks, _CH)
    dst = dst.reshape(_NW, nchunks, _CH)

    deg_all = _make_deg(npad, nchunks)(dst)
    mp = _make_mp(npad, nchunks, d)

    xp = jnp.pad(x, ((0, npad - n), (0, 0)))
    p1 = mp(xp, src, dst)
    h1 = _gcn_dense(p1, deg_all, W0, b0.reshape(1, d))
    p2 = mp(h1, src, dst)
    pi, v = _heads(
        n, p2, deg_all, b1.reshape(1, d), W1,
        Wpg, Wpd, Wv,
        bpg.reshape(1, 1), bpd.reshape(1, 1), bv.reshape(1, 1),
    )
    return (pi, v)
